# Initial kernel scaffold; baseline (speedup 1.0000x reference)
import jax
import jax.numpy as jnp
from jax.experimental import pallas as pl


def kernel(atom_type_ids, sh_coeffs):
    raise NotImplementedError("write your pallas kernel here")


# speedup vs baseline: 3.4766x; 3.4766x over previous
import functools

import jax
import jax.numpy as jnp
from jax import lax
from jax.experimental import pallas as pl
from jax.experimental.pallas import tpu as pltpu
from jax.experimental.pallas import tpu_sc as plsc

N_ATOMS = 2_000_000
NUM_TYPES = 118
D = 48
L = 128
G = 8
CHUNK = G * L
NW = 32
FULL_CHUNKS = N_ATOMS // (CHUNK * NW)
TAIL_START = FULL_CHUNKS * CHUNK * NW
TAIL = N_ATOMS - TAIL_START


def _sc_gather(ids, table):
  mesh = plsc.VectorSubcoreMesh(core_axis_name="c", subcore_axis_name="s")

  @functools.partial(
      pl.kernel,
      out_type=jax.ShapeDtypeStruct((N_ATOMS, D), jnp.float32),
      mesh=mesh,
      scratch_types=[
          pltpu.VMEM((CHUNK,), jnp.int32),
          pltpu.VMEM((CHUNK, D), jnp.float32),
          pltpu.SemaphoreType.DMA,
      ],
      compiler_params=pltpu.CompilerParams(use_tc_tiling_on_sc=False),
  )
  def k(ids_hbm, table_hbm, out_hbm, idx_v, rows_v, sem):
    wid = lax.axis_index("s") * 2 + lax.axis_index("c")
    base = wid * FULL_CHUNKS * CHUNK

    def do_chunk(off, n_groups):
      pltpu.sync_copy(ids_hbm.at[pl.ds(off, n_groups * L)],
                      idx_v.at[pl.ds(0, n_groups * L)])
      descs = [
          pltpu.async_copy(
              table_hbm.at[idx_v.at[pl.ds(g * L, L)]],
              rows_v.at[pl.ds(g * L, L)], sem)
          for g in range(n_groups)
      ]
      for d in descs:
        d.wait()
      pltpu.sync_copy(rows_v.at[pl.ds(0, n_groups * L)],
                      out_hbm.at[pl.ds(off, n_groups * L)])

    def chunk_body(t, carry):
      do_chunk(base + t * CHUNK, G)
      return carry

    lax.fori_loop(0, FULL_CHUNKS, chunk_body, 0)

    @pl.when(wid == NW - 1)
    def _tail():
      do_chunk(TAIL_START, G)
      do_chunk(TAIL_START + CHUNK, (TAIL - CHUNK) // L)

  return k(ids, table)


def kernel(atom_type_ids, sh_coeffs):
  ids = atom_type_ids.astype(jnp.int32)
  table = sh_coeffs.reshape(NUM_TYPES, D)
  out = _sc_gather(ids, table)
  return out.reshape(N_ATOMS, 16, 3)


# speedup vs baseline: 13.3830x; 3.8495x over previous
import functools

import jax
import jax.numpy as jnp
from jax import lax
from jax.experimental import pallas as pl
from jax.experimental.pallas import tpu as pltpu
from jax.experimental.pallas import tpu_sc as plsc

N_ATOMS = 2_000_000
NUM_TYPES = 118
NC = 3
NK = 16
NP = NC * NK
CHUNK = 1024
NW = 32
FULL_CHUNKS = N_ATOMS // (CHUNK * NW)
TAIL_START = FULL_CHUNKS * CHUNK * NW
TAIL = N_ATOMS - TAIL_START


def _sc_gather(ids, table_flat):
  mesh = plsc.VectorSubcoreMesh(core_axis_name="c", subcore_axis_name="s")

  @functools.partial(
      pl.kernel,
      out_type=jax.ShapeDtypeStruct((NC, NK, N_ATOMS), jnp.float32),
      mesh=mesh,
      scratch_types=[
          pltpu.VMEM((NP * NUM_TYPES,), jnp.float32),
          pltpu.VMEM((CHUNK,), jnp.int32),
          pltpu.VMEM((NC, NK, CHUNK), jnp.float32),
      ],
      compiler_params=pltpu.CompilerParams(
          use_tc_tiling_on_sc=True, needs_layout_passes=False),
  )
  def k(ids_hbm, table_hbm, out_hbm, table_v, idx_v, tchunk, sem=None):
    wid = lax.axis_index("s") * 2 + lax.axis_index("c")
    base = wid * FULL_CHUNKS * CHUNK
    pltpu.sync_copy(table_hbm, table_v)

    def compute_group(g, n_atoms_16):
      ids16 = idx_v[pl.ds(g * 16, 16)]
      for p in range(NP):
        vals = plsc.load_gather(table_v, [ids16 + p * NUM_TYPES])
        tchunk[p // NK, p % NK, pl.ds(g * 16, 16)] = vals
      return n_atoms_16

    def do_chunk(off, n_atoms):
      pltpu.sync_copy(ids_hbm.at[pl.ds(off, n_atoms)],
                      idx_v.at[pl.ds(0, n_atoms)])
      lax.fori_loop(0, n_atoms // 16, compute_group, 0)
      pltpu.sync_copy(tchunk.at[:, :, pl.ds(0, n_atoms)],
                      out_hbm.at[:, :, pl.ds(off, n_atoms)])

    def chunk_body(t, carry):
      do_chunk(base + t * CHUNK, CHUNK)
      return carry

    lax.fori_loop(0, FULL_CHUNKS, chunk_body, 0)

    @pl.when(wid == NW - 1)
    def _tail():
      do_chunk(TAIL_START, CHUNK)
      do_chunk(TAIL_START + CHUNK, TAIL - CHUNK)

  return k(ids, table_flat)


def kernel(atom_type_ids, sh_coeffs):
  ids = atom_type_ids.astype(jnp.int32)
  table_flat = jnp.transpose(sh_coeffs, (2, 1, 0)).reshape(NP * NUM_TYPES)
  out = _sc_gather(ids, table_flat)
  return jnp.transpose(out, (2, 1, 0))


# speedup vs baseline: 38.6626x; 2.8889x over previous
import functools

import jax
import jax.numpy as jnp
from jax import lax
from jax.experimental import pallas as pl
from jax.experimental.pallas import tpu as pltpu
from jax.experimental.pallas import tpu_sc as plsc

N_ATOMS = 2_000_000
NUM_TYPES = 118
NC = 3
NK = 16
NP = NC * NK
CHUNK = 1024
NW = 32
FULL_CHUNKS = N_ATOMS // (CHUNK * NW)
TAIL_START = FULL_CHUNKS * CHUNK * NW
TAIL = N_ATOMS - TAIL_START


def _sc_gather(ids, table_flat):
  mesh = plsc.VectorSubcoreMesh(core_axis_name="c", subcore_axis_name="s")

  @functools.partial(
      pl.kernel,
      out_type=jax.ShapeDtypeStruct((NC, NK, N_ATOMS), jnp.float32),
      mesh=mesh,
      scratch_types=[
          pltpu.VMEM((NP * NUM_TYPES,), jnp.float32),
          pltpu.VMEM((CHUNK,), jnp.int32),
          pltpu.VMEM((NC, NK, CHUNK), jnp.float32),
      ],
      compiler_params=pltpu.CompilerParams(
          use_tc_tiling_on_sc=True, needs_layout_passes=False),
  )
  def k(ids_hbm, table_hbm, out_hbm, table_v, idx_v, tchunk, sem=None):
    wid = lax.axis_index("s") * 2 + lax.axis_index("c")
    base = wid * FULL_CHUNKS * CHUNK
    pltpu.sync_copy(table_hbm, table_v)

    def do_chunk(off, n_atoms):
      pltpu.sync_copy(ids_hbm.at[pl.ds(off, n_atoms)],
                      idx_v.at[pl.ds(0, n_atoms)])

      @plsc.parallel_loop(0, n_atoms // 16, unroll=2)
      def compute_group(g):
        ids16 = idx_v[pl.ds(g * 16, 16)]
        for p in range(NP):
          vals = plsc.load_gather(table_v, [ids16 + p * NUM_TYPES])
          tchunk[p // NK, p % NK, pl.ds(g * 16, 16)] = vals
      pltpu.sync_copy(tchunk.at[:, :, pl.ds(0, n_atoms)],
                      out_hbm.at[:, :, pl.ds(off, n_atoms)])

    def chunk_body(t, carry):
      do_chunk(base + t * CHUNK, CHUNK)
      return carry

    lax.fori_loop(0, FULL_CHUNKS, chunk_body, 0)

    @pl.when(wid == NW - 1)
    def _tail():
      do_chunk(TAIL_START, CHUNK)
      do_chunk(TAIL_START + CHUNK, TAIL - CHUNK)

  return k(ids, table_flat)


def kernel(atom_type_ids, sh_coeffs):
  ids = atom_type_ids.astype(jnp.int32)
  table_flat = jnp.transpose(sh_coeffs, (2, 1, 0)).reshape(NP * NUM_TYPES)
  out = _sc_gather(ids, table_flat)
  return jnp.transpose(out, (2, 1, 0))
